# baseline (speedup 1.0000x reference)
import jax
import jax.numpy as jnp
from jax.experimental import pallas as pl
from jax.experimental.pallas import tpu as pltpu

EPS = 0.001
SUPPORT = 300.0
BINS = 601
BINS_PAD = 640
ROWS_PER_BLOCK = 2048


def _twohot_block(scalar_ref, out_ref):
    x = scalar_ref[0, 0, :]
    x = jnp.where(jnp.isnan(x) | jnp.isinf(x), 0.0, x)
    t = jnp.sign(x) * (jnp.sqrt(jnp.abs(x) + 1.0) - 1.0) + EPS * x
    shifted = jnp.clip(t, -SUPPORT, SUPPORT) + SUPPORT
    colf = jax.lax.broadcasted_iota(
        jnp.int32, (x.shape[0], BINS_PAD), 1
    ).astype(jnp.float32)
    out_ref[:, :] = jnp.maximum(1.0 - jnp.abs(shifted[:, None] - colf), 0.0)


def _twohot_padded(chunk):
    m = chunk.shape[0]
    nblocks = m // ROWS_PER_BLOCK
    chunk3d = chunk.reshape(nblocks, 1, ROWS_PER_BLOCK)
    return pl.pallas_call(
        _twohot_block,
        grid=(nblocks,),
        in_specs=[pl.BlockSpec((1, 1, ROWS_PER_BLOCK), lambda i: (i, 0, 0))],
        out_specs=pl.BlockSpec((ROWS_PER_BLOCK, BINS_PAD), lambda i: (i, 0)),
        out_shape=jax.ShapeDtypeStruct((m, BINS_PAD), jnp.float32),
        compiler_params=pltpu.CompilerParams(
            dimension_semantics=("arbitrary",),
        ),
    )(chunk3d)


@jax.jit
def _twohot(scalar):
    n = scalar.shape[0]
    half = n // 2
    top = _twohot_padded(scalar[:half])[:, :BINS]
    bot = _twohot_padded(scalar[half:])[:, :BINS]
    return jnp.concatenate([top, bot], axis=0)


def kernel(scalar, support_size):
    return _twohot(scalar)


# speedup vs baseline: 1.6220x; 1.6220x over previous
import jax
import jax.numpy as jnp
from jax.experimental import pallas as pl
from jax.experimental.pallas import tpu as pltpu

EPS = 0.001
SUPPORT = 300.0
BINS = 601
BINS_PAD = 640
ROWS_PER_BLOCK = 2048


def _twohot_block(scalar_ref, out_ref):
    x = scalar_ref[0, 0, :]
    x = jnp.where(jnp.isnan(x) | jnp.isinf(x), 0.0, x)
    t = jnp.sign(x) * (jnp.sqrt(jnp.abs(x) + 1.0) - 1.0) + EPS * x
    shifted = jnp.clip(t, -SUPPORT, SUPPORT) + SUPPORT
    colf = jax.lax.broadcasted_iota(
        jnp.int32, (x.shape[0], BINS_PAD), 1
    ).astype(jnp.float32)
    out_ref[:, :] = jnp.maximum(1.0 - jnp.abs(shifted[:, None] - colf), 0.0)


def _twohot_padded(chunk):
    m = chunk.shape[0]
    nblocks = m // ROWS_PER_BLOCK
    chunk3d = chunk.reshape(nblocks, 1, ROWS_PER_BLOCK)
    return pl.pallas_call(
        _twohot_block,
        grid=(nblocks,),
        in_specs=[pl.BlockSpec((1, 1, ROWS_PER_BLOCK), lambda i: (i, 0, 0))],
        out_specs=pl.BlockSpec((ROWS_PER_BLOCK, BINS_PAD), lambda i: (i, 0)),
        out_shape=jax.ShapeDtypeStruct((m, BINS_PAD), jnp.float32),
        compiler_params=pltpu.CompilerParams(
            dimension_semantics=("parallel",),
        ),
    )(chunk3d)


@jax.jit
def _twohot(scalar):
    return _twohot_padded(scalar)[:, :BINS]


def kernel(scalar, support_size):
    return _twohot(scalar)
